# baseline (speedup 1.0000x reference)
import functools

import jax
import jax.numpy as jnp
from jax import lax
from jax.experimental import pallas as pl
from jax.experimental.pallas import tpu as pltpu
from jax.experimental.pallas import tpu_sc as plsc

NN = 10000
NP = 10240
EE = 320000
EROWS = 2560
EPAD = EROWS * 128
EMB_D = 64
HID = 128
NG = 16
NCLS = 32
NSC = 2
VOC = 100000


def _sc_deg_emb_body(col_hbm, x_hbm, emb2_hbm, deg_out, h_out,
                     deg_sh, cidx, ones_v, stage_v, xidx, rows_v, rows_v2, sem, sem2):
    c = lax.axis_index("c")
    s = lax.axis_index("s")

    @pl.when(c == 0)
    def _():
        z16 = jnp.zeros((16,), jnp.float32)
        o16 = jnp.ones((16,), jnp.float32)

        def zb(i, carry):
            stage_v[pl.ds(i * 16, 16)] = z16
            return carry

        lax.fori_loop(0, 40, zb, 0)

        def ob(i, carry):
            ones_v[pl.ds(i * 16, 16)] = o16
            return carry

        lax.fori_loop(0, 8, ob, 0)
        pltpu.sync_copy(stage_v, deg_sh.at[pl.ds(s * 640, 640)])

    plsc.subcore_barrier()

    @pl.when(c == 0)
    def _():
        pltpu.sync_copy(col_hbm.at[pl.ds(s * 160, 160)], cidx)

        def body(j, carry):
            pltpu.sync_copy(ones_v, deg_sh.at[cidx.at[j]], add=True)
            return carry

        lax.fori_loop(0, 160, body, 0)

    @pl.when(c == 1)
    def _():
        pltpu.sync_copy(x_hbm.at[pl.ds(s * 8, 8)], xidx)
        pltpu.async_copy(emb2_hbm.at[xidx.at[0]], rows_v, sem)
        pltpu.async_copy(emb2_hbm.at[xidx.at[1]], rows_v2, sem2)

        def body(i, carry):
            j2 = i * 2
            pltpu.make_async_copy(emb2_hbm.at[pl.ds(0, 80)], rows_v, sem).wait()
            pltpu.sync_copy(rows_v, h_out.at[pl.ds(s * 640 + j2 * 80, 80)])

            @pl.when(j2 + 2 < 8)
            def _():
                pltpu.async_copy(emb2_hbm.at[xidx.at[j2 + 2]], rows_v, sem)

            pltpu.make_async_copy(emb2_hbm.at[pl.ds(0, 80)], rows_v2, sem2).wait()
            pltpu.sync_copy(rows_v2, h_out.at[pl.ds(s * 640 + (j2 + 1) * 80, 80)])

            @pl.when(j2 + 3 < 8)
            def _():
                pltpu.async_copy(emb2_hbm.at[xidx.at[j2 + 3]], rows_v2, sem2)

            return carry

        lax.fori_loop(0, 4, body, 0)

    plsc.subcore_barrier()

    @pl.when(c == 0)
    def _():
        pltpu.sync_copy(deg_sh.at[pl.ds(s * 640, 640)], stage_v)
        pltpu.sync_copy(stage_v, deg_out.at[pl.ds(s * 640, 640)])


def _sc_scatter_body(xs_hbm, row_hbm, col_hbm, z_hbm, out_hbm, acc_sh,
                     ridx, cidx, m0, m1, sem0, sem1):
    c = lax.axis_index("c")
    s = lax.axis_index("s")
    bufs = (m0, m1)
    sems = (sem0, sem1)

    pltpu.sync_copy(z_hbm, acc_sh.at[pl.ds(s * 640, 640)])

    base = (c * 16 + s) * 80
    plsc.subcore_barrier()

    for half in range(2):
        hb = base + half * 40
        pltpu.sync_copy(row_hbm.at[pl.ds(hb, 40)], ridx)
        pltpu.sync_copy(col_hbm.at[pl.ds(hb, 40)], cidx)
        for b in range(2):
            pltpu.async_copy(xs_hbm.at[ridx.at[b]], bufs[b], sems[b])

        def body(i, carry):
            j2 = i * 2
            for b in range(2):
                j = j2 + b
                pltpu.make_async_copy(xs_hbm.at[pl.ds(0, 128)], bufs[b], sems[b]).wait()
                pltpu.sync_copy(bufs[b], acc_sh.at[cidx.at[j]], add=True)

                @pl.when(j + 2 < 40)
                def _():
                    pltpu.async_copy(xs_hbm.at[ridx.at[j + 2]], bufs[b], sems[b])
            return carry

        lax.fori_loop(0, 20, body, 0)

    plsc.subcore_barrier()
    pltpu.sync_copy(acc_sh.at[pl.ds(s * 640, 640)],
                    out_hbm.at[pl.ds((c * 16 + s) * 640, 640)])


@functools.cache
def _sc_kernels():
    mesh = plsc.VectorSubcoreMesh(core_axis_name="c", subcore_axis_name="s")
    deg_emb = pl.kernel(
        _sc_deg_emb_body,
        mesh=mesh,
        out_type=(
            jax.ShapeDtypeStruct((NP,), jnp.float32),
            jax.ShapeDtypeStruct((NP, 128), jnp.float32),
        ),
        scratch_types=[
            pltpu.VMEM_SHARED((NP,), jnp.float32),
            pltpu.VMEM((160, 128), jnp.int32),
            pltpu.VMEM((128,), jnp.float32),
            pltpu.VMEM((640,), jnp.float32),
            pltpu.VMEM((8, 80), jnp.int32),
            pltpu.VMEM((80, 128), jnp.float32),
            pltpu.VMEM((80, 128), jnp.float32),
            pltpu.SemaphoreType.DMA,
            pltpu.SemaphoreType.DMA,
        ],
    )
    scatter = pl.kernel(
        _sc_scatter_body,
        mesh=mesh,
        out_type=jax.ShapeDtypeStruct((NSC * NP, HID), jnp.float32),
        scratch_types=[
            pltpu.VMEM_SHARED((NP, HID), jnp.float32),
            pltpu.VMEM((40, 128), jnp.int32),
            pltpu.VMEM((40, 128), jnp.int32),
            pltpu.VMEM((128, HID), jnp.float32),
            pltpu.VMEM((128, HID), jnp.float32),
            pltpu.SemaphoreType.DMA,
            pltpu.SemaphoreType.DMA,
        ],
    )
    return deg_emb, scatter


_GRID = 8
_RB = NP // _GRID


def _tc_lin1_body(g, par, deg, w2x, xs_o, dinv_o):
    lane = lax.broadcasted_iota(jnp.int32, (_RB, 128), 1)
    keep = jnp.where((lane < EMB_D) == (par[...] == 0), 1.0, 0.0)
    dinv = lax.rsqrt(deg[...] + 1.0)
    xs_o[...] = jnp.dot(g[...] * keep, w2x[...],
                        preferred_element_type=jnp.float32) * dinv
    dinv_o[...] = dinv


def _tc_lin1(g, par, deg, w2x):
    return pl.pallas_call(
        _tc_lin1_body,
        grid=(_GRID,),
        in_specs=[
            pl.BlockSpec((_RB, 128), lambda i: (i, 0)),
            pl.BlockSpec((_RB, 1), lambda i: (i, 0)),
            pl.BlockSpec((_RB, 1), lambda i: (i, 0)),
            pl.BlockSpec((2 * EMB_D, HID), lambda i: (0, 0)),
        ],
        out_specs=[
            pl.BlockSpec((_RB, HID), lambda i: (i, 0)),
            pl.BlockSpec((_RB, 1), lambda i: (i, 0)),
        ],
        out_shape=[
            jax.ShapeDtypeStruct((NP, HID), jnp.float32),
            jax.ShapeDtypeStruct((NP, 1), jnp.float32),
        ],
    )(g, par, deg, w2x)


def _tc_mid_body(p, xs, dinv, w, b, xs2_o):
    h1 = jnp.maximum((p[0] + p[1] + xs[...]) * dinv[...] + b[...], 0.0)
    xs2_o[...] = jnp.dot(h1, w[...], preferred_element_type=jnp.float32) * dinv[...]


def _tc_mid(p, xs, dinv, w, b):
    return pl.pallas_call(
        _tc_mid_body,
        grid=(_GRID,),
        in_specs=[
            pl.BlockSpec((NSC, _RB, HID), lambda i: (0, i, 0)),
            pl.BlockSpec((_RB, HID), lambda i: (i, 0)),
            pl.BlockSpec((_RB, 1), lambda i: (i, 0)),
            pl.BlockSpec((HID, HID), lambda i: (0, 0)),
            pl.BlockSpec((1, HID), lambda i: (0, 0)),
        ],
        out_specs=pl.BlockSpec((_RB, HID), lambda i: (i, 0)),
        out_shape=jax.ShapeDtypeStruct((NP, HID), jnp.float32),
    )(p, xs, dinv, w, b)


def _tc_fin_body(p, xs, dinv, b, bat, wl, bl_, o):
    h2 = jnp.maximum((p[0] + p[1] + xs[...]) * dinv[...] + b[...], 0.0)
    gids = lax.broadcasted_iota(jnp.int32, (NG, NP), 0)
    onehot = (bat[...] == gids).astype(jnp.float32)
    sums = jnp.dot(onehot, h2, preferred_element_type=jnp.float32)
    cnt = jnp.sum(onehot, axis=1, keepdims=True)
    pooled = sums / jnp.maximum(cnt, 1.0)
    o[...] = jnp.dot(pooled, wl[...], preferred_element_type=jnp.float32) + bl_[...]


def _tc_fin(p, xs, dinv, b, bat, wl, bl_):
    return pl.pallas_call(
        _tc_fin_body,
        out_shape=jax.ShapeDtypeStruct((NG, NCLS), jnp.float32),
    )(p, xs, dinv, b, bat, wl, bl_)


def kernel(x, edge_index, batch, emb, W1, b1, W2, b2, Wl, bl):
    x = x.astype(jnp.int32)
    ei = edge_index.astype(jnp.int32)
    row = jnp.concatenate([ei[0], jnp.zeros((EPAD - EE,), jnp.int32)])
    pad_dst = NN + (jnp.arange(EPAD - EE, dtype=jnp.int32) % (NP - NN))
    col = jnp.concatenate([ei[1], pad_dst])
    row2d = row.reshape(EROWS, 128)
    col2d = col.reshape(EROWS, 128)
    xp = jnp.concatenate([x, jnp.zeros((NP - NN,), jnp.int32)])
    xpair = (xp >> 1).reshape(NP // 80, 80)
    xpar = (xp & 1).reshape(NP, 1)
    batp = jnp.concatenate(
        [batch.astype(jnp.int32), jnp.full((NP - NN,), NG, jnp.int32)]
    ).reshape(1, NP)
    emb2 = emb.reshape(VOC // 2, 128)
    w2x = jnp.concatenate([W1, W1], axis=0)

    zeros640 = jnp.zeros((640, HID), jnp.float32)
    _sc_deg_emb, _sc_scatter = _sc_kernels()
    deg, g = _sc_deg_emb(col2d, xpair, emb2)
    deg = deg.reshape(NP, 1)
    xs1, dinv = _tc_lin1(g, xpar, deg, w2x)
    p1 = _sc_scatter(xs1, row2d, col2d, zeros640).reshape(NSC, NP, HID)
    xs2 = _tc_mid(p1, xs1, dinv, W2, b1.reshape(1, HID))
    p2 = _sc_scatter(xs2, row2d, col2d, zeros640).reshape(NSC, NP, HID)
    return _tc_fin(p2, xs2, dinv, b2.reshape(1, HID), batp, Wl, bl.reshape(1, NCLS))


# speedup vs baseline: 1.1466x; 1.1466x over previous
import functools

import jax
import jax.numpy as jnp
from jax import lax
from jax.experimental import pallas as pl
from jax.experimental.pallas import tpu as pltpu
from jax.experimental.pallas import tpu_sc as plsc

NN = 10000
NP = 10240
EE = 320000
EROWS = 2560
EPAD = EROWS * 128
EMB_D = 64
HID = 128
NG = 16
NCLS = 32
NSC = 2
VOC = 100000


def _sc_deg_emb_body(col_hbm, x_hbm, emb2_hbm, deg_out, h_out,
                     deg_sh, cidx, ones_v, stage_v, xidx, rows_v, rows_v2, sem, sem2):
    c = lax.axis_index("c")
    s = lax.axis_index("s")

    @pl.when(c == 0)
    def _():
        z16 = jnp.zeros((16,), jnp.float32)
        o16 = jnp.ones((16,), jnp.float32)

        def zb(i, carry):
            stage_v[pl.ds(i * 16, 16)] = z16
            return carry

        lax.fori_loop(0, 40, zb, 0)

        def ob(i, carry):
            ones_v[pl.ds(i * 16, 16)] = o16
            return carry

        lax.fori_loop(0, 8, ob, 0)
        pltpu.sync_copy(stage_v, deg_sh.at[pl.ds(s * 640, 640)])

    plsc.subcore_barrier()

    @pl.when(c == 0)
    def _():
        pltpu.sync_copy(col_hbm.at[pl.ds(s * 160, 160)], cidx)

        def body(j, carry):
            pltpu.sync_copy(ones_v, deg_sh.at[cidx.at[j]], add=True)
            return carry

        lax.fori_loop(0, 160, body, 0)

    @pl.when(c == 1)
    def _():
        pltpu.sync_copy(x_hbm.at[pl.ds(s * 8, 8)], xidx)
        pltpu.async_copy(emb2_hbm.at[xidx.at[0]], rows_v, sem)
        pltpu.async_copy(emb2_hbm.at[xidx.at[1]], rows_v2, sem2)

        def body(i, carry):
            j2 = i * 2
            pltpu.make_async_copy(emb2_hbm.at[pl.ds(0, 80)], rows_v, sem).wait()
            pltpu.sync_copy(rows_v, h_out.at[pl.ds(s * 640 + j2 * 80, 80)])

            @pl.when(j2 + 2 < 8)
            def _():
                pltpu.async_copy(emb2_hbm.at[xidx.at[j2 + 2]], rows_v, sem)

            pltpu.make_async_copy(emb2_hbm.at[pl.ds(0, 80)], rows_v2, sem2).wait()
            pltpu.sync_copy(rows_v2, h_out.at[pl.ds(s * 640 + (j2 + 1) * 80, 80)])

            @pl.when(j2 + 3 < 8)
            def _():
                pltpu.async_copy(emb2_hbm.at[xidx.at[j2 + 3]], rows_v2, sem2)

            return carry

        lax.fori_loop(0, 4, body, 0)

    plsc.subcore_barrier()

    @pl.when(c == 0)
    def _():
        pltpu.sync_copy(deg_sh.at[pl.ds(s * 640, 640)], stage_v)
        pltpu.sync_copy(stage_v, deg_out.at[pl.ds(s * 640, 640)])


def _sc_scatter_body(xs_hbm, row_hbm, col_hbm, z_hbm, out_hbm, acc_sh,
                     ridx, cidx, m0, m1, m2, m3,
                     g0, g1, g2, g3, s0, s1, s2, s3):
    c = lax.axis_index("c")
    s = lax.axis_index("s")
    bufs = (m0, m1, m2, m3)
    gsem = (g0, g1, g2, g3)
    ssem = (s0, s1, s2, s3)

    pltpu.sync_copy(z_hbm, acc_sh.at[pl.ds(s * 640, 640)])

    base = (c * 16 + s) * 160
    plsc.subcore_barrier()

    def gissue(j, b):
        pltpu.async_copy(xs_hbm.at[ridx.at[j]], bufs[b], gsem[b])

    def gwait(b):
        pltpu.make_async_copy(xs_hbm.at[pl.ds(0, 64)], bufs[b], gsem[b]).wait()

    def sissue(j, b):
        pltpu.async_copy(bufs[b], acc_sh.at[cidx.at[j]], ssem[b], add=True)

    def swait(b):
        pltpu.make_async_copy(bufs[b], acc_sh.at[pl.ds(0, 64)], ssem[b]).wait()

    for qtr in range(4):
        hb = base + qtr * 40
        pltpu.sync_copy(row_hbm.at[pl.ds(hb, 40)], ridx)
        pltpu.sync_copy(col_hbm.at[pl.ds(hb, 40)], cidx)
        for t in range(4):
            gissue(t, t)

        def body(k, carry):
            j0 = k * 4
            for t in range(4):
                gwait(t)
                sissue(j0 + t, t)
            for t in range(4):
                swait(t)
                gissue(j0 + 4 + t, t)
            return carry

        lax.fori_loop(0, 9, body, 0)

        for t in range(4):
            gwait(t)
            sissue(36 + t, t)
        for t in range(4):
            swait(t)

    plsc.subcore_barrier()
    pltpu.sync_copy(acc_sh.at[pl.ds(s * 640, 640)],
                    out_hbm.at[pl.ds((c * 16 + s) * 640, 640)])


@functools.cache
def _sc_kernels():
    mesh = plsc.VectorSubcoreMesh(core_axis_name="c", subcore_axis_name="s")
    deg_emb = pl.kernel(
        _sc_deg_emb_body,
        mesh=mesh,
        out_type=(
            jax.ShapeDtypeStruct((NP,), jnp.float32),
            jax.ShapeDtypeStruct((NP, 128), jnp.float32),
        ),
        scratch_types=[
            pltpu.VMEM_SHARED((NP,), jnp.float32),
            pltpu.VMEM((160, 128), jnp.int32),
            pltpu.VMEM((128,), jnp.float32),
            pltpu.VMEM((640,), jnp.float32),
            pltpu.VMEM((8, 80), jnp.int32),
            pltpu.VMEM((80, 128), jnp.float32),
            pltpu.VMEM((80, 128), jnp.float32),
            pltpu.SemaphoreType.DMA,
            pltpu.SemaphoreType.DMA,
        ],
    )
    scatter = pl.kernel(
        _sc_scatter_body,
        mesh=mesh,
        out_type=jax.ShapeDtypeStruct((NSC * NP, HID), jnp.float32),
        scratch_types=[
            pltpu.VMEM_SHARED((NP, HID), jnp.float32),
            pltpu.VMEM((40, 64), jnp.int32),
            pltpu.VMEM((40, 64), jnp.int32),
        ]
        + [pltpu.VMEM((64, HID), jnp.float32) for _ in range(4)]
        + [pltpu.SemaphoreType.DMA for _ in range(8)],
    )
    return deg_emb, scatter


_GRID = 8
_RB = NP // _GRID


def _tc_lin1_body(g, par, deg, w2x, xs_o, dinv_o):
    lane = lax.broadcasted_iota(jnp.int32, (_RB, 128), 1)
    keep = jnp.where((lane < EMB_D) == (par[...] == 0), 1.0, 0.0)
    dinv = lax.rsqrt(deg[...] + 1.0)
    xs_o[...] = jnp.dot(g[...] * keep, w2x[...],
                        preferred_element_type=jnp.float32) * dinv
    dinv_o[...] = dinv


def _tc_lin1(g, par, deg, w2x):
    return pl.pallas_call(
        _tc_lin1_body,
        grid=(_GRID,),
        in_specs=[
            pl.BlockSpec((_RB, 128), lambda i: (i, 0)),
            pl.BlockSpec((_RB, 1), lambda i: (i, 0)),
            pl.BlockSpec((_RB, 1), lambda i: (i, 0)),
            pl.BlockSpec((2 * EMB_D, HID), lambda i: (0, 0)),
        ],
        out_specs=[
            pl.BlockSpec((_RB, HID), lambda i: (i, 0)),
            pl.BlockSpec((_RB, 1), lambda i: (i, 0)),
        ],
        out_shape=[
            jax.ShapeDtypeStruct((NP, HID), jnp.float32),
            jax.ShapeDtypeStruct((NP, 1), jnp.float32),
        ],
    )(g, par, deg, w2x)


def _tc_mid_body(p, xs, dinv, w, b, xs2_o):
    h1 = jnp.maximum((p[0] + p[1] + xs[...]) * dinv[...] + b[...], 0.0)
    xs2_o[...] = jnp.dot(h1, w[...], preferred_element_type=jnp.float32) * dinv[...]


def _tc_mid(p, xs, dinv, w, b):
    return pl.pallas_call(
        _tc_mid_body,
        grid=(_GRID,),
        in_specs=[
            pl.BlockSpec((NSC, _RB, HID), lambda i: (0, i, 0)),
            pl.BlockSpec((_RB, HID), lambda i: (i, 0)),
            pl.BlockSpec((_RB, 1), lambda i: (i, 0)),
            pl.BlockSpec((HID, HID), lambda i: (0, 0)),
            pl.BlockSpec((1, HID), lambda i: (0, 0)),
        ],
        out_specs=pl.BlockSpec((_RB, HID), lambda i: (i, 0)),
        out_shape=jax.ShapeDtypeStruct((NP, HID), jnp.float32),
    )(p, xs, dinv, w, b)


def _tc_fin_body(p, xs, dinv, b, bat, wl, bl_, o):
    h2 = jnp.maximum((p[0] + p[1] + xs[...]) * dinv[...] + b[...], 0.0)
    gids = lax.broadcasted_iota(jnp.int32, (NG, NP), 0)
    onehot = (bat[...] == gids).astype(jnp.float32)
    sums = jnp.dot(onehot, h2, preferred_element_type=jnp.float32)
    cnt = jnp.sum(onehot, axis=1, keepdims=True)
    pooled = sums / jnp.maximum(cnt, 1.0)
    o[...] = jnp.dot(pooled, wl[...], preferred_element_type=jnp.float32) + bl_[...]


def _tc_fin(p, xs, dinv, b, bat, wl, bl_):
    return pl.pallas_call(
        _tc_fin_body,
        out_shape=jax.ShapeDtypeStruct((NG, NCLS), jnp.float32),
    )(p, xs, dinv, b, bat, wl, bl_)


def kernel(x, edge_index, batch, emb, W1, b1, W2, b2, Wl, bl):
    x = x.astype(jnp.int32)
    ei = edge_index.astype(jnp.int32)
    row = jnp.concatenate([ei[0], jnp.zeros((EPAD - EE,), jnp.int32)])
    pad_dst = NN + (jnp.arange(EPAD - EE, dtype=jnp.int32) % (NP - NN))
    col = jnp.concatenate([ei[1], pad_dst])
    row2d = row.reshape(EROWS, 128)
    col2d = col.reshape(EROWS, 128)
    row64 = row.reshape(EROWS * 2, 64)
    col64 = col.reshape(EROWS * 2, 64)
    xp = jnp.concatenate([x, jnp.zeros((NP - NN,), jnp.int32)])
    xpair = (xp >> 1).reshape(NP // 80, 80)
    xpar = (xp & 1).reshape(NP, 1)
    batp = jnp.concatenate(
        [batch.astype(jnp.int32), jnp.full((NP - NN,), NG, jnp.int32)]
    ).reshape(1, NP)
    emb2 = emb.reshape(VOC // 2, 128)
    w2x = jnp.concatenate([W1, W1], axis=0)

    zeros640 = jnp.zeros((640, HID), jnp.float32)
    _sc_deg_emb, _sc_scatter = _sc_kernels()
    deg, g = _sc_deg_emb(col2d, xpair, emb2)
    deg = deg.reshape(NP, 1)
    xs1, dinv = _tc_lin1(g, xpar, deg, w2x)
    p1 = _sc_scatter(xs1, row64, col64, zeros640).reshape(NSC, NP, HID)
    xs2 = _tc_mid(p1, xs1, dinv, W2, b1.reshape(1, HID))
    p2 = _sc_scatter(xs2, row64, col64, zeros640).reshape(NSC, NP, HID)
    return _tc_fin(p2, xs2, dinv, b2.reshape(1, HID), batp, Wl, bl.reshape(1, NCLS))
